# Initial kernel scaffold; baseline (speedup 1.0000x reference)
import jax
import jax.numpy as jnp
from jax.experimental import pallas as pl


def kernel(u, it, A, B, shiftA, shiftB, bu, bi, mu):
    raise NotImplementedError("write your pallas kernel here")


# speedup vs baseline: 3.1284x; 3.1284x over previous
import functools

import jax
import jax.numpy as jnp
from jax import lax
from jax.experimental import pallas as pl
from jax.experimental.pallas import tpu as pltpu
from jax.experimental.pallas import tpu_sc as plsc

_NC, _NS, _L = 2, 16, 16
_NW = _NC * _NS
_CH = 128


def _sc_gather(u, it, A, B, sA_flat, sB_flat, bu, bi):
    Bn = u.shape[0]
    D = A.shape[1]
    NU = bu.shape[0]
    NI = bi.shape[0]
    K = sA_flat.shape[0] // NU
    bpw = Bn // _NW
    nch = bpw // _CH
    mesh = plsc.VectorSubcoreMesh(core_axis_name="c", subcore_axis_name="s")

    @functools.partial(
        pl.kernel,
        out_type=(
            jax.ShapeDtypeStruct((Bn, D), jnp.float32),
            jax.ShapeDtypeStruct((K, Bn), jnp.float32),
            jax.ShapeDtypeStruct((K, Bn), jnp.float32),
            jax.ShapeDtypeStruct((Bn,), jnp.float32),
        ),
        mesh=mesh,
        scratch_types=[
            pltpu.VMEM((bpw,), jnp.int32),
            pltpu.VMEM((bpw,), jnp.int32),
            pltpu.VMEM((_CH,), jnp.int32),
            pltpu.VMEM((_CH, 128), jnp.float32),
            pltpu.VMEM((_CH, 128), jnp.float32),
            pltpu.VMEM((_CH,), jnp.float32),
            pltpu.VMEM((4 * bpw,), jnp.float32),
            pltpu.VMEM((4 * bpw,), jnp.float32),
            pltpu.VMEM((bpw,), jnp.float32),
        ],
    )
    def body(u_hbm, it_hbm, A_hbm, B_hbm, sA_hbm, sB_hbm, bu_hbm, bi_hbm,
             ab_out, sa_out, sb_out, bias_out,
             uidx, iidx, sidx, bufA, bufB, sland, sabuf, sbbuf, biasbuf):
        wid = lax.axis_index("s") * _NC + lax.axis_index("c")
        base = wid * bpw
        pltpu.sync_copy(u_hbm.at[pl.ds(base, bpw)], uidx)
        pltpu.sync_copy(it_hbm.at[pl.ds(base, bpw)], iidx)

        for c in range(nch):
            pltpu.sync_copy(A_hbm.at[uidx.at[pl.ds(c * _CH, _CH)]], bufA)
            pltpu.sync_copy(B_hbm.at[iidx.at[pl.ds(c * _CH, _CH)]], bufB)

            def prod(i, _):
                for j in range(D // _L):
                    sl = pl.ds(j * _L, _L)
                    bufA[i, sl] = bufA[i, sl] * bufB[i, sl]
                return 0
            lax.fori_loop(0, _CH, prod, 0)
            pltpu.sync_copy(bufA, ab_out.at[pl.ds(base + c * _CH, _CH), :])

        for tabs in range(2):
            idxr = uidx if tabs == 0 else iidx
            tab = sA_hbm if tabs == 0 else sB_hbm
            buf = sabuf if tabs == 0 else sbbuf
            N = NU if tabs == 0 else NI
            for k in range(K):
                for c in range(nch):
                    def mkidx(j, _):
                        sl = pl.ds(j * _L, _L)
                        sidx[sl] = idxr[pl.ds(c * _CH + j * _L, _L)] + k * N
                        return 0
                    lax.fori_loop(0, _CH // _L, mkidx, 0)
                    pltpu.sync_copy(tab.at[sidx],
                                    buf.at[pl.ds(k * bpw + c * _CH, _CH)])
        for k in range(K):
            pltpu.sync_copy(sabuf.at[pl.ds(k * bpw, bpw)],
                            sa_out.at[k, pl.ds(base, bpw)])
            pltpu.sync_copy(sbbuf.at[pl.ds(k * bpw, bpw)],
                            sb_out.at[k, pl.ds(base, bpw)])

        for c in range(nch):
            pltpu.sync_copy(bu_hbm.at[uidx.at[pl.ds(c * _CH, _CH)]], sland)

            def cpb(j, _):
                sl = pl.ds(c * _CH + j * _L, _L)
                biasbuf[sl] = sland[pl.ds(j * _L, _L)]
                return 0
            lax.fori_loop(0, _CH // _L, cpb, 0)
            pltpu.sync_copy(bi_hbm.at[iidx.at[pl.ds(c * _CH, _CH)]], sland)

            def addb(j, _):
                sl = pl.ds(c * _CH + j * _L, _L)
                biasbuf[sl] = biasbuf[sl] + sland[pl.ds(j * _L, _L)]
                return 0
            lax.fori_loop(0, _CH // _L, addb, 0)
        pltpu.sync_copy(biasbuf, bias_out.at[pl.ds(base, bpw)])

    return body(u, it, A, B, sA_flat, sB_flat, bu, bi)


def _tc_combine(ab, sa, sb, bias, mu_arr, K):
    Bn, D = ab.shape
    M = 2048 if Bn % 2048 == 0 else Bn
    half = D / 2.0
    Kf = float(K)

    def body(mu_ref, ab_ref, sa_ref, sb_ref, bias_ref, out_ref):
        AB = ab_ref[:, :]
        r2 = 2.0 * lax.broadcasted_iota(jnp.int32, (D, D), 0).astype(jnp.float32)
        j = lax.broadcasted_iota(jnp.int32, (D, D), 1)
        jm = ((j % K).astype(jnp.float32) + 1.0) / Kf
        W = jnp.where(j < K, jnp.cos(r2 * jm),
                      jnp.where(j < 2 * K, jnp.sin(r2 * jm),
                                jnp.where(j == 2 * K, 1.0, 0.0)))
        dots = jnp.dot(AB, W, preferred_element_type=jnp.float32,
                       precision=lax.Precision.HIGHEST)
        S0 = dots[:, 2 * K]
        t = jnp.zeros((M,), jnp.float32)
        for k in range(K):
            omk = (k + 1.0) / Kf
            sak = half * sa_ref[k, :]
            sbk = half * sb_ref[k, :]
            t = (t + jnp.cos((sak - sbk) * omk) * S0
                 - jnp.cos((sak + sbk) * omk) * dots[:, k]
                 - jnp.sin((sak + sbk) * omk) * dots[:, K + k])
        out_ref[:] = t * (1.0 / (2.0 * Kf * Kf)) + bias_ref[:] + mu_ref[0]

    return pl.pallas_call(
        body,
        grid=(Bn // M,),
        in_specs=[
            pl.BlockSpec(memory_space=pltpu.SMEM),
            pl.BlockSpec((M, D), lambda i: (i, 0)),
            pl.BlockSpec((K, M), lambda i: (0, i)),
            pl.BlockSpec((K, M), lambda i: (0, i)),
            pl.BlockSpec((M,), lambda i: (i,)),
        ],
        out_specs=pl.BlockSpec((M,), lambda i: (i,)),
        out_shape=jax.ShapeDtypeStruct((Bn,), jnp.float32),
    )(mu_arr, ab, sa, sb, bias)


def kernel(u, it, A, B, shiftA, shiftB, bu, bi, mu):
    K = shiftA.shape[0]
    ab, sa, sb, bias = _sc_gather(
        u.astype(jnp.int32), it.astype(jnp.int32), A, B,
        shiftA.reshape(-1), shiftB.reshape(-1), bu, bi)
    return _tc_combine(ab, sa, sb, bias, jnp.reshape(mu, (1,)), K)


# speedup vs baseline: 3.2838x; 1.0497x over previous
import functools

import jax
import jax.numpy as jnp
from jax import lax
from jax.experimental import pallas as pl
from jax.experimental.pallas import tpu as pltpu
from jax.experimental.pallas import tpu_sc as plsc

_NC, _NS, _L = 2, 16, 16
_NW = _NC * _NS
_CH = 128


def _sc_gather(u, it, A, B, sA_flat, sB_flat, bu, bi):
    Bn = u.shape[0]
    D = A.shape[1]
    NU = bu.shape[0]
    NI = bi.shape[0]
    K = sA_flat.shape[0] // NU
    bpw = Bn // _NW
    nch = bpw // _CH
    mesh = plsc.VectorSubcoreMesh(core_axis_name="c", subcore_axis_name="s")

    @functools.partial(
        pl.kernel,
        out_type=(
            jax.ShapeDtypeStruct((Bn, D), jnp.float32),
            jax.ShapeDtypeStruct((K, Bn), jnp.float32),
            jax.ShapeDtypeStruct((K, Bn), jnp.float32),
            jax.ShapeDtypeStruct((Bn,), jnp.float32),
        ),
        mesh=mesh,
        scratch_types=[
            pltpu.VMEM((bpw,), jnp.int32),
            pltpu.VMEM((bpw,), jnp.int32),
            pltpu.VMEM((_CH,), jnp.int32),
            pltpu.VMEM((_CH, 128), jnp.float32),
            pltpu.VMEM((_CH, 128), jnp.float32),
            pltpu.VMEM((_CH,), jnp.float32),
            pltpu.VMEM((4 * bpw,), jnp.float32),
            pltpu.VMEM((4 * bpw,), jnp.float32),
            pltpu.VMEM((bpw,), jnp.float32),
        ],
    )
    def body(u_hbm, it_hbm, A_hbm, B_hbm, sA_hbm, sB_hbm, bu_hbm, bi_hbm,
             ab_out, sa_out, sb_out, bias_out,
             uidx, iidx, sidx, bufA, bufB, sland, sabuf, sbbuf, biasbuf):
        wid = lax.axis_index("s") * _NC + lax.axis_index("c")
        base = wid * bpw
        pltpu.sync_copy(u_hbm.at[pl.ds(base, bpw)], uidx)
        pltpu.sync_copy(it_hbm.at[pl.ds(base, bpw)], iidx)

        for c in range(nch):
            pltpu.sync_copy(A_hbm.at[uidx.at[pl.ds(c * _CH, _CH)]], bufA)
            pltpu.sync_copy(B_hbm.at[iidx.at[pl.ds(c * _CH, _CH)]], bufB)

            def prod(i, _):
                for j in range(D // _L):
                    sl = pl.ds(j * _L, _L)
                    bufA[i, sl] = bufA[i, sl] * bufB[i, sl]
                return 0
            lax.fori_loop(0, _CH, prod, 0)
            pltpu.sync_copy(bufA, ab_out.at[pl.ds(base + c * _CH, _CH), :])

        for tabs in range(2):
            idxr = uidx if tabs == 0 else iidx
            tab = sA_hbm if tabs == 0 else sB_hbm
            buf = sabuf if tabs == 0 else sbbuf
            N = NU if tabs == 0 else NI
            for k in range(K):
                for c in range(nch):
                    def mkidx(j, _):
                        sl = pl.ds(j * _L, _L)
                        sidx[sl] = idxr[pl.ds(c * _CH + j * _L, _L)] + k * N
                        return 0
                    lax.fori_loop(0, _CH // _L, mkidx, 0)
                    pltpu.sync_copy(tab.at[sidx],
                                    buf.at[pl.ds(k * bpw + c * _CH, _CH)])
        for k in range(K):
            pltpu.sync_copy(sabuf.at[pl.ds(k * bpw, bpw)],
                            sa_out.at[k, pl.ds(base, bpw)])
            pltpu.sync_copy(sbbuf.at[pl.ds(k * bpw, bpw)],
                            sb_out.at[k, pl.ds(base, bpw)])

        for c in range(nch):
            pltpu.sync_copy(bu_hbm.at[uidx.at[pl.ds(c * _CH, _CH)]], sland)

            def cpb(j, _):
                sl = pl.ds(c * _CH + j * _L, _L)
                biasbuf[sl] = sland[pl.ds(j * _L, _L)]
                return 0
            lax.fori_loop(0, _CH // _L, cpb, 0)
            pltpu.sync_copy(bi_hbm.at[iidx.at[pl.ds(c * _CH, _CH)]], sland)

            def addb(j, _):
                sl = pl.ds(c * _CH + j * _L, _L)
                biasbuf[sl] = biasbuf[sl] + sland[pl.ds(j * _L, _L)]
                return 0
            lax.fori_loop(0, _CH // _L, addb, 0)
        pltpu.sync_copy(biasbuf, bias_out.at[pl.ds(base, bpw)])

    return body(u, it, A, B, sA_flat, sB_flat, bu, bi)


def _tc_combine(ab, sa, sb, bias, mu_arr, K):
    Bn, D = ab.shape
    M = 2048 if Bn % 2048 == 0 else Bn
    half = D / 2.0
    Kf = float(K)

    NW = 16

    def body(mu_ref, ab_ref, sa_ref, sb_ref, bias_ref, out_ref):
        AB = ab_ref[:, :]
        r2 = 2.0 * lax.broadcasted_iota(jnp.int32, (D, NW), 0).astype(jnp.float32)
        j = lax.broadcasted_iota(jnp.int32, (D, NW), 1)
        jm = ((j % K).astype(jnp.float32) + 1.0) / Kf
        W = jnp.where(j < K, jnp.cos(r2 * jm),
                      jnp.where(j < 2 * K, jnp.sin(r2 * jm),
                                jnp.where(j == 2 * K, 1.0, 0.0)))
        dots = jnp.dot(AB, W, preferred_element_type=jnp.float32,
                       precision=lax.Precision.HIGHEST)
        jg = lax.broadcasted_iota(jnp.int32, (M, NW), 1)
        cd1 = jnp.zeros((M,), jnp.float32)
        G = jnp.zeros((M, NW), jnp.float32)
        for k in range(K):
            omk = (k + 1.0) / Kf
            sak = half * sa_ref[k, :]
            sbk = half * sb_ref[k, :]
            cd1 = cd1 + jnp.cos((sak - sbk) * omk)
            d2 = (sak + sbk) * omk
            G = jnp.where(jg == k, -jnp.cos(d2)[:, None], G)
            G = jnp.where(jg == K + k, -jnp.sin(d2)[:, None], G)
        G = jnp.where(jg == 2 * K, cd1[:, None], G)
        t = jnp.sum(dots * G, axis=1)
        out_ref[:] = t * (1.0 / (2.0 * Kf * Kf)) + bias_ref[:] + mu_ref[0]

    return pl.pallas_call(
        body,
        grid=(Bn // M,),
        in_specs=[
            pl.BlockSpec(memory_space=pltpu.SMEM),
            pl.BlockSpec((M, D), lambda i: (i, 0)),
            pl.BlockSpec((K, M), lambda i: (0, i)),
            pl.BlockSpec((K, M), lambda i: (0, i)),
            pl.BlockSpec((M,), lambda i: (i,)),
        ],
        out_specs=pl.BlockSpec((M,), lambda i: (i,)),
        out_shape=jax.ShapeDtypeStruct((Bn,), jnp.float32),
    )(mu_arr, ab, sa, sb, bias)


def kernel(u, it, A, B, shiftA, shiftB, bu, bi, mu):
    K = shiftA.shape[0]
    ab, sa, sb, bias = _sc_gather(
        u.astype(jnp.int32), it.astype(jnp.int32), A, B,
        shiftA.reshape(-1), shiftB.reshape(-1), bu, bi)
    return _tc_combine(ab, sa, sb, bias, jnp.reshape(mu, (1,)), K)


# speedup vs baseline: 3.3115x; 1.0084x over previous
import functools

import jax
import jax.numpy as jnp
from jax import lax
from jax.experimental import pallas as pl
from jax.experimental.pallas import tpu as pltpu
from jax.experimental.pallas import tpu_sc as plsc

_NC, _NS, _L = 2, 16, 16
_NW = _NC * _NS
_CH = 128


def _sc_gather(u, it, A, B, sA_flat, sB_flat, bu, bi):
    Bn = u.shape[0]
    D = A.shape[1]
    NU = bu.shape[0]
    NI = bi.shape[0]
    K = sA_flat.shape[0] // NU
    KP = 4 * K
    bpw = Bn // _NW
    nch = bpw // _CH
    half = D / 2.0
    mesh = plsc.VectorSubcoreMesh(core_axis_name="c", subcore_axis_name="s")

    @functools.partial(
        pl.kernel,
        out_type=(
            jax.ShapeDtypeStruct((Bn, D), jnp.float32),
            jax.ShapeDtypeStruct((KP, Bn), jnp.float32),
            jax.ShapeDtypeStruct((Bn,), jnp.float32),
        ),
        mesh=mesh,
        scratch_types=[
            pltpu.VMEM((bpw,), jnp.int32),
            pltpu.VMEM((bpw,), jnp.int32),
            pltpu.VMEM((_CH,), jnp.int32),
            pltpu.VMEM((_CH, 128), jnp.float32),
            pltpu.VMEM((_CH, 128), jnp.float32),
            pltpu.VMEM((_CH,), jnp.float32),
            pltpu.VMEM((4 * bpw,), jnp.float32),
            pltpu.VMEM((4 * bpw,), jnp.float32),
            pltpu.VMEM((16 * bpw,), jnp.float32),
            pltpu.VMEM((bpw,), jnp.float32),
        ],
    )
    def body(u_hbm, it_hbm, A_hbm, B_hbm, sA_hbm, sB_hbm, bu_hbm, bi_hbm,
             ab_out, ph_out, bias_out,
             uidx, iidx, sidx, bufA, bufB, sland, sabuf, sbbuf, phbuf, biasbuf):
        wid = lax.axis_index("s") * _NC + lax.axis_index("c")
        base = wid * bpw
        pltpu.sync_copy(u_hbm.at[pl.ds(base, bpw)], uidx)
        pltpu.sync_copy(it_hbm.at[pl.ds(base, bpw)], iidx)

        for c in range(nch):
            pltpu.sync_copy(A_hbm.at[uidx.at[pl.ds(c * _CH, _CH)]], bufA)
            pltpu.sync_copy(B_hbm.at[iidx.at[pl.ds(c * _CH, _CH)]], bufB)

            def prod(i, _):
                for j in range(D // _L):
                    sl = pl.ds(j * _L, _L)
                    bufA[i, sl] = bufA[i, sl] * bufB[i, sl]
                return 0
            lax.fori_loop(0, _CH, prod, 0)
            pltpu.sync_copy(bufA, ab_out.at[pl.ds(base + c * _CH, _CH), :])

        for tabs in range(2):
            idxr = uidx if tabs == 0 else iidx
            tab = sA_hbm if tabs == 0 else sB_hbm
            buf = sabuf if tabs == 0 else sbbuf
            N = NU if tabs == 0 else NI
            for k in range(K):
                for c in range(nch):
                    def mkidx(j, _):
                        sl = pl.ds(j * _L, _L)
                        sidx[sl] = idxr[pl.ds(c * _CH + j * _L, _L)] + k * N
                        return 0
                    lax.fori_loop(0, _CH // _L, mkidx, 0)
                    pltpu.sync_copy(tab.at[sidx],
                                    buf.at[pl.ds(k * bpw + c * _CH, _CH)])

        for k in range(K):
            sc = (k + 1.0) / K * half

            def mkph(j, _):
                sl = pl.ds(k * bpw + j * _L, _L)
                a = sabuf[sl] * sc
                b = sbbuf[sl] * sc
                phbuf[pl.ds(k * bpw + j * _L, _L)] = a - b
                d2 = a + b
                phbuf[pl.ds((K + k) * bpw + j * _L, _L)] = d2
                phbuf[pl.ds((2 * K + k) * bpw + j * _L, _L)] = d2
                phbuf[pl.ds((3 * K + k) * bpw + j * _L, _L)] = jnp.zeros(
                    (_L,), jnp.float32)
                return 0
            lax.fori_loop(0, bpw // _L, mkph, 0)
        for r in range(KP):
            pltpu.sync_copy(phbuf.at[pl.ds(r * bpw, bpw)],
                            ph_out.at[r, pl.ds(base, bpw)])

        for c in range(nch):
            pltpu.sync_copy(bu_hbm.at[uidx.at[pl.ds(c * _CH, _CH)]], sland)

            def cpb(j, _):
                sl = pl.ds(c * _CH + j * _L, _L)
                biasbuf[sl] = sland[pl.ds(j * _L, _L)]
                return 0
            lax.fori_loop(0, _CH // _L, cpb, 0)
            pltpu.sync_copy(bi_hbm.at[iidx.at[pl.ds(c * _CH, _CH)]], sland)

            def addb(j, _):
                sl = pl.ds(c * _CH + j * _L, _L)
                biasbuf[sl] = biasbuf[sl] + sland[pl.ds(j * _L, _L)]
                return 0
            lax.fori_loop(0, _CH // _L, addb, 0)
        pltpu.sync_copy(biasbuf, bias_out.at[pl.ds(base, bpw)])

    return body(u, it, A, B, sA_flat, sB_flat, bu, bi)


def _tc_combine(ab, ph, bias, mu_arr, K):
    Bn, D = ab.shape
    KP = ph.shape[0]
    M = 2048 if Bn % 2048 == 0 else Bn
    Kf = float(K)
    PI = 3.14159265358979323846

    def body(mu_ref, ab_ref, ph_ref, bias_ref, out_ref):
        AB = ab_ref[:, :]
        jr = lax.broadcasted_iota(jnp.int32, (KP, 1), 0)
        off = jnp.where(jr < K, 0.0, jnp.where(jr < 2 * K, PI, PI * 0.5))
        coefT = jnp.cos(ph_ref[:, :] + off)
        j2 = lax.broadcasted_iota(jnp.int32, (KP, D), 0)
        r2 = 2.0 * lax.broadcasted_iota(jnp.int32, (KP, D), 1).astype(
            jnp.float32)
        omj = ((j2 % K).astype(jnp.float32) + 1.0) / Kf
        W2 = jnp.where(j2 < K, 1.0,
                       jnp.where(j2 < 2 * K, jnp.cos(r2 * omj),
                                 jnp.where(j2 < 3 * K, jnp.sin(r2 * omj),
                                           0.0)))
        Q = lax.dot_general(coefT, W2, (((0,), (0,)), ((), ())),
                            preferred_element_type=jnp.float32,
                            precision=lax.Precision.HIGHEST)
        t = jnp.sum(AB * Q, axis=1)
        out_ref[:] = t * (1.0 / (2.0 * Kf * Kf)) + bias_ref[:] + mu_ref[0]

    return pl.pallas_call(
        body,
        grid=(Bn // M,),
        in_specs=[
            pl.BlockSpec(memory_space=pltpu.SMEM),
            pl.BlockSpec((M, D), lambda i: (i, 0)),
            pl.BlockSpec((KP, M), lambda i: (0, i)),
            pl.BlockSpec((M,), lambda i: (i,)),
        ],
        out_specs=pl.BlockSpec((M,), lambda i: (i,)),
        out_shape=jax.ShapeDtypeStruct((Bn,), jnp.float32),
    )(mu_arr, ab, ph, bias)


def kernel(u, it, A, B, shiftA, shiftB, bu, bi, mu):
    K = shiftA.shape[0]
    ab, ph, bias = _sc_gather(
        u.astype(jnp.int32), it.astype(jnp.int32), A, B,
        shiftA.reshape(-1), shiftB.reshape(-1), bu, bi)
    return _tc_combine(ab, ph, bias, jnp.reshape(mu, (1,)), K)


# speedup vs baseline: 6.7391x; 2.0351x over previous
import functools

import jax
import jax.numpy as jnp
from jax import lax
from jax.experimental import pallas as pl
from jax.experimental.pallas import tpu as pltpu
from jax.experimental.pallas import tpu_sc as plsc

_NC, _NS, _L = 2, 16, 16
_NW = _NC * _NS
_RCH = 64
_SCH = 128


def _sc_gather(u, it, A, B, sA_flat, sB_flat, bu, bi):
    Bn = u.shape[0]
    D = A.shape[1]
    NU = bu.shape[0]
    NI = bi.shape[0]
    K = sA_flat.shape[0] // NU
    KP = 4 * K
    bpw = Bn // _NW
    nrch = bpw // _RCH
    nsch = bpw // _SCH
    half = D / 2.0
    mesh = plsc.VectorSubcoreMesh(core_axis_name="c", subcore_axis_name="s")

    @functools.partial(
        pl.kernel,
        out_type=(
            jax.ShapeDtypeStruct((Bn, D), jnp.float32),
            jax.ShapeDtypeStruct((KP, Bn), jnp.float32),
            jax.ShapeDtypeStruct((Bn,), jnp.float32),
        ),
        mesh=mesh,
        scratch_types=[
            pltpu.VMEM((bpw,), jnp.int32),
            pltpu.VMEM((bpw,), jnp.int32),
            pltpu.VMEM((2 * K, bpw), jnp.int32),
            pltpu.VMEM((2, _RCH, 128), jnp.float32),
            pltpu.VMEM((2, _RCH, 128), jnp.float32),
            pltpu.VMEM((2, _RCH, 128), jnp.float32),
            pltpu.VMEM((2 * K, bpw), jnp.float32),
            pltpu.VMEM((2, bpw), jnp.float32),
            pltpu.VMEM((bpw,), jnp.float32),
            pltpu.VMEM((bpw,), jnp.float32),
            pltpu.SemaphoreType.DMA,
            pltpu.SemaphoreType.DMA,
            pltpu.SemaphoreType.DMA,
            pltpu.SemaphoreType.DMA,
            pltpu.SemaphoreType.DMA,
        ],
    )
    def body(u_hbm, it_hbm, A_hbm, B_hbm, sA_hbm, sB_hbm, bu_hbm, bi_hbm,
             ab_out, ph_out, bias_out,
             uidx, iidx, sidx, bufA, bufB, bufP, shbuf, bland, pht1, pht2,
             semS, semA, semB, semW, semP):
        wid = lax.axis_index("s") * _NC + lax.axis_index("c")
        base = wid * bpw
        pltpu.sync_copy(u_hbm.at[pl.ds(base, bpw)], uidx)
        pltpu.sync_copy(it_hbm.at[pl.ds(base, bpw)], iidx)

        for t in range(2):
            idxr = uidx if t == 0 else iidx
            N = NU if t == 0 else NI
            for k in range(K):
                def mkidx(j, _):
                    sl = pl.ds(j * _L, _L)
                    sidx[t * K + k, sl] = idxr[sl] + k * N
                    return 0
                lax.fori_loop(0, bpw // _L, mkidx, 0)

        sdescs = []
        for t in range(2):
            tab = sA_hbm if t == 0 else sB_hbm
            for k in range(K):
                for c in range(nsch):
                    d = pltpu.make_async_copy(
                        tab.at[sidx.at[t * K + k, pl.ds(c * _SCH, _SCH)]],
                        shbuf.at[t * K + k, pl.ds(c * _SCH, _SCH)],
                        semS)
                    d.start()
                    sdescs.append(d)
        for t in range(2):
            tab = bu_hbm if t == 0 else bi_hbm
            idxr = uidx if t == 0 else iidx
            for c in range(nsch):
                d = pltpu.make_async_copy(
                    tab.at[idxr.at[pl.ds(c * _SCH, _SCH)]],
                    bland.at[t, pl.ds(c * _SCH, _SCH)], semS)
                d.start()
                sdescs.append(d)

        def fire_row(c):
            s = c % 2
            da = pltpu.make_async_copy(
                A_hbm.at[uidx.at[pl.ds(c * _RCH, _RCH)]], bufA.at[s], semA)
            db = pltpu.make_async_copy(
                B_hbm.at[iidx.at[pl.ds(c * _RCH, _RCH)]], bufB.at[s], semB)
            da.start()
            db.start()
            return da, db

        rdescs = {}
        rdescs[0] = fire_row(0)
        if nrch > 1:
            rdescs[1] = fire_row(1)
        wdescs = {}
        for c in range(nrch):
            s = c % 2
            da, db = rdescs.pop(c)
            da.wait()
            db.wait()
            if c - 2 in wdescs:
                wdescs.pop(c - 2).wait()

            def prod(i, _):
                for j in range(D // _L):
                    sl = pl.ds(j * _L, _L)
                    bufP[s, i, sl] = bufA[s, i, sl] * bufB[s, i, sl]
                return 0
            lax.fori_loop(0, _RCH, prod, 0)
            dw = pltpu.make_async_copy(
                bufP.at[s], ab_out.at[pl.ds(base + c * _RCH, _RCH), :], semW)
            dw.start()
            wdescs[c] = dw
            if c + 2 < nrch:
                rdescs[c + 2] = fire_row(c + 2)

        for d in sdescs:
            d.wait()

        for k in range(K):
            sc_ = (k + 1.0) / K * half

            def mkph(j, _):
                sl = pl.ds(j * _L, _L)
                a = shbuf[k, sl] * sc_
                b = shbuf[K + k, sl] * sc_
                pht1[sl] = a - b
                pht2[sl] = a + b
                return 0
            lax.fori_loop(0, bpw // _L, mkph, 0)
            pds = []
            for row, src in ((k, pht1), (K + k, pht2), (2 * K + k, pht2)):
                d = pltpu.make_async_copy(
                    src, ph_out.at[row, pl.ds(base, bpw)], semP)
                d.start()
                pds.append(d)
            for d in pds:
                d.wait()

        def zrow(j, _):
            pht1[pl.ds(j * _L, _L)] = jnp.zeros((_L,), jnp.float32)
            return 0
        lax.fori_loop(0, bpw // _L, zrow, 0)
        zds = []
        for k in range(K):
            d = pltpu.make_async_copy(
                pht1, ph_out.at[3 * K + k, pl.ds(base, bpw)], semP)
            d.start()
            zds.append(d)

        def mkbias(j, _):
            sl = pl.ds(j * _L, _L)
            pht2[sl] = bland[0, sl] + bland[1, sl]
            return 0
        lax.fori_loop(0, bpw // _L, mkbias, 0)
        pltpu.sync_copy(pht2, bias_out.at[pl.ds(base, bpw)])
        for d in zds:
            d.wait()

        for c in sorted(wdescs):
            wdescs.pop(c).wait()

    return body(u, it, A, B, sA_flat, sB_flat, bu, bi)


def _tc_combine(ab, ph, bias, mu_arr, K):
    Bn, D = ab.shape
    KP = ph.shape[0]
    M = 2048 if Bn % 2048 == 0 else Bn
    Kf = float(K)
    PI = 3.14159265358979323846

    def body(mu_ref, ab_ref, ph_ref, bias_ref, out_ref):
        AB = ab_ref[:, :]
        jr = lax.broadcasted_iota(jnp.int32, (KP, 1), 0)
        off = jnp.where(jr < K, 0.0, jnp.where(jr < 2 * K, PI, PI * 0.5))
        coefT = jnp.cos(ph_ref[:, :] + off)
        j2 = lax.broadcasted_iota(jnp.int32, (KP, D), 0)
        r2 = 2.0 * lax.broadcasted_iota(jnp.int32, (KP, D), 1).astype(
            jnp.float32)
        omj = ((j2 % K).astype(jnp.float32) + 1.0) / Kf
        W2 = jnp.where(j2 < K, 1.0,
                       jnp.where(j2 < 2 * K, jnp.cos(r2 * omj),
                                 jnp.where(j2 < 3 * K, jnp.sin(r2 * omj),
                                           0.0)))
        dots = lax.dot_general(W2, AB, (((1,), (1,)), ((), ())),
                               preferred_element_type=jnp.float32,
                               precision=lax.Precision.HIGHEST)
        t = jnp.sum(coefT * dots, axis=0)
        out_ref[:] = t * (1.0 / (2.0 * Kf * Kf)) + bias_ref[:] + mu_ref[0]

    return pl.pallas_call(
        body,
        grid=(Bn // M,),
        in_specs=[
            pl.BlockSpec(memory_space=pltpu.SMEM),
            pl.BlockSpec((M, D), lambda i: (i, 0)),
            pl.BlockSpec((KP, M), lambda i: (0, i)),
            pl.BlockSpec((M,), lambda i: (i,)),
        ],
        out_specs=pl.BlockSpec((M,), lambda i: (i,)),
        out_shape=jax.ShapeDtypeStruct((Bn,), jnp.float32),
    )(mu_arr, ab, ph, bias)


def kernel(u, it, A, B, shiftA, shiftB, bu, bi, mu):
    K = shiftA.shape[0]
    ab, ph, bias = _sc_gather(
        u.astype(jnp.int32), it.astype(jnp.int32), A, B,
        shiftA.reshape(-1), shiftB.reshape(-1), bu, bi)
    return _tc_combine(ab, ph, bias, jnp.reshape(mu, (1,)), K)


# speedup vs baseline: 6.9878x; 1.0369x over previous
import functools

import jax
import jax.numpy as jnp
from jax import lax
from jax.experimental import pallas as pl
from jax.experimental.pallas import tpu as pltpu
from jax.experimental.pallas import tpu_sc as plsc

_NC, _NS, _L = 2, 16, 16
_NW = _NC * _NS
_RCH = 128
_SCH = 128


def _sc_gather(u, it, A, B, sA_flat, sB_flat, bu, bi, K):
    Bn = u.shape[0]
    D = A.shape[1]
    NU = sA_flat.shape[0] // K
    NI = sB_flat.shape[0] // K
    KP = 4 * K
    bpw = Bn // _NW
    nrch = bpw // _RCH
    nsch = bpw // _SCH
    half = D / 2.0
    mesh = plsc.VectorSubcoreMesh(core_axis_name="c", subcore_axis_name="s")

    @functools.partial(
        pl.kernel,
        out_type=(
            jax.ShapeDtypeStruct((Bn, D), jnp.float32),
            jax.ShapeDtypeStruct((KP, Bn), jnp.float32),
            jax.ShapeDtypeStruct((Bn,), jnp.float32),
        ),
        mesh=mesh,
        scratch_types=[
            pltpu.VMEM((bpw,), jnp.int32),
            pltpu.VMEM((bpw,), jnp.int32),
            pltpu.VMEM((2 * K, bpw), jnp.int32),
            pltpu.VMEM((2, _RCH, 128), jnp.float32),
            pltpu.VMEM((2, _RCH, 128), jnp.float32),
            pltpu.VMEM((2, _RCH, 128), jnp.float32),
            pltpu.VMEM((2 * K, bpw), jnp.float32),
            pltpu.VMEM((2, bpw), jnp.float32),
            pltpu.VMEM((3 * K, bpw), jnp.float32),
            pltpu.VMEM((bpw,), jnp.float32),
            pltpu.SemaphoreType.DMA,
            pltpu.SemaphoreType.DMA,
            pltpu.SemaphoreType.DMA,
            pltpu.SemaphoreType.DMA,
            pltpu.SemaphoreType.DMA,
        ],
    )
    def body(u_hbm, it_hbm, A_hbm, B_hbm, sA_hbm, sB_hbm, bu_hbm, bi_hbm,
             ab_out, ph_out, bias_out,
             uidx, iidx, sidx, bufA, bufB, bufP, shbuf, bland, phbuf, ztmp,
             semS, semA, semB, semW, semP):
        wid = lax.axis_index("s") * _NC + lax.axis_index("c")
        base = wid * bpw
        pltpu.sync_copy(u_hbm.at[pl.ds(base, bpw)], uidx)
        pltpu.sync_copy(it_hbm.at[pl.ds(base, bpw)], iidx)

        for t in range(2):
            idxr = uidx if t == 0 else iidx
            N = NU if t == 0 else NI

            def mkidx(p, _):
                k = p >> 5
                j = p & 31
                sl = pl.ds(j * _L, _L)
                sidx[t * K + k, sl] = idxr[sl] + k * N
                return 0
            lax.fori_loop(0, K * (bpw // _L), mkidx, 0)
        for t in range(2):
            tab = sA_hbm if t == 0 else sB_hbm

            def fire_scalar(p, _):
                k = p >> 2
                c = p & 3
                pltpu.make_async_copy(
                    tab.at[sidx.at[t * K + k, pl.ds(c * _SCH, _SCH)]],
                    shbuf.at[t * K + k, pl.ds(c * _SCH, _SCH)],
                    semS).start()
                return 0
            lax.fori_loop(0, K * nsch, fire_scalar, 0)
        for t in range(2):
            tab = bu_hbm if t == 0 else bi_hbm
            idxr = uidx if t == 0 else iidx

            def fire_bias(c, _):
                pltpu.make_async_copy(
                    tab.at[idxr.at[pl.ds(c * _SCH, _SCH)]],
                    bland.at[t, pl.ds(c * _SCH, _SCH)], semS).start()
                return 0
            lax.fori_loop(0, nsch, fire_bias, 0)

        def fire_row(c):
            s = c % 2
            da = pltpu.make_async_copy(
                A_hbm.at[uidx.at[pl.ds(c * _RCH, _RCH)]], bufA.at[s], semA)
            db = pltpu.make_async_copy(
                B_hbm.at[iidx.at[pl.ds(c * _RCH, _RCH)]], bufB.at[s], semB)
            da.start()
            db.start()
            return da, db

        rdescs = {}
        rdescs[0] = fire_row(0)
        if nrch > 1:
            rdescs[1] = fire_row(1)
        wdescs = {}
        for c in range(nrch):
            s = c % 2
            da, db = rdescs.pop(c)
            da.wait()
            db.wait()
            if c - 2 in wdescs:
                wdescs.pop(c - 2).wait()

            def prod(i, _):
                for j in range(D // _L):
                    sl = pl.ds(j * _L, _L)
                    bufP[s, i, sl] = bufA[s, i, sl] * bufB[s, i, sl]
                return 0
            lax.fori_loop(0, _RCH, prod, 0)
            dw = pltpu.make_async_copy(
                bufP.at[s], ab_out.at[pl.ds(base + c * _RCH, _RCH), :], semW)
            dw.start()
            wdescs[c] = dw
            if c + 2 < nrch:
                rdescs[c + 2] = fire_row(c + 2)

        for t in range(2):
            tab = sA_hbm if t == 0 else sB_hbm

            def drain_scalar(p, _):
                k = p >> 2
                c = p & 3
                pltpu.make_async_copy(
                    tab.at[sidx.at[t * K + k, pl.ds(c * _SCH, _SCH)]],
                    shbuf.at[t * K + k, pl.ds(c * _SCH, _SCH)],
                    semS).wait()
                return 0
            lax.fori_loop(0, K * nsch, drain_scalar, 0)
        for t in range(2):
            tab = bu_hbm if t == 0 else bi_hbm
            idxr = uidx if t == 0 else iidx

            def drain_bias(c, _):
                pltpu.make_async_copy(
                    tab.at[idxr.at[pl.ds(c * _SCH, _SCH)]],
                    bland.at[t, pl.ds(c * _SCH, _SCH)], semS).wait()
                return 0
            lax.fori_loop(0, nsch, drain_bias, 0)

        def mkph(k, _):
            sc_ = (k.astype(jnp.float32) + 1.0) * (half / K)

            def inner(j, _2):
                sl = pl.ds(j * _L, _L)
                a = shbuf[k, sl] * sc_
                b = shbuf[K + k, sl] * sc_
                phbuf[k, sl] = a - b
                phbuf[K + k, sl] = a + b
                phbuf[2 * K + k, sl] = a + b
                return 0
            lax.fori_loop(0, bpw // _L, inner, 0)
            return 0
        lax.fori_loop(0, K, mkph, 0)

        def fire_ph(r, _):
            pltpu.make_async_copy(phbuf.at[r],
                                  ph_out.at[r, pl.ds(base, bpw)],
                                  semP).start()
            return 0
        lax.fori_loop(0, 3 * K, fire_ph, 0)

        def zrow(j, _):
            ztmp[pl.ds(j * _L, _L)] = jnp.zeros((_L,), jnp.float32)
            return 0
        lax.fori_loop(0, bpw // _L, zrow, 0)

        def fire_z(k, _):
            pltpu.make_async_copy(ztmp,
                                  ph_out.at[3 * K + k, pl.ds(base, bpw)],
                                  semP).start()
            return 0
        lax.fori_loop(0, K, fire_z, 0)

        def mkbias(j, _):
            sl = pl.ds(j * _L, _L)
            bland[0, sl] = bland[0, sl] + bland[1, sl]
            return 0
        lax.fori_loop(0, bpw // _L, mkbias, 0)
        dbias = pltpu.make_async_copy(bland.at[0],
                                      bias_out.at[pl.ds(base, bpw)], semP)
        dbias.start()

        def drain_ph(r, _):
            pltpu.make_async_copy(phbuf.at[0],
                                  ph_out.at[r, pl.ds(base, bpw)],
                                  semP).wait()
            return 0
        lax.fori_loop(0, KP, drain_ph, 0)
        dbias.wait()
        for c in sorted(wdescs):
            wdescs.pop(c).wait()

    return body(u, it, A, B, sA_flat, sB_flat, bu, bi)


def _tc_combine(ab, ph, bias, mu_arr, K):
    Bn, D = ab.shape
    KP = ph.shape[0]
    M = 2048 if Bn % 2048 == 0 else Bn
    Kf = float(K)
    PI = 3.14159265358979323846

    def body(mu_ref, ab_ref, ph_ref, bias_ref, out_ref):
        AB = ab_ref[:, :]
        jr = lax.broadcasted_iota(jnp.int32, (KP, 1), 0)
        off = jnp.where(jr < K, 0.0, jnp.where(jr < 2 * K, PI, PI * 0.5))
        coefT = jnp.cos(ph_ref[:, :] + off)
        j2 = lax.broadcasted_iota(jnp.int32, (KP, D), 0)
        r2 = 2.0 * lax.broadcasted_iota(jnp.int32, (KP, D), 1).astype(
            jnp.float32)
        omj = ((j2 % K).astype(jnp.float32) + 1.0) / Kf
        W2 = jnp.where(j2 < K, 1.0,
                       jnp.where(j2 < 2 * K, jnp.cos(r2 * omj),
                                 jnp.where(j2 < 3 * K, jnp.sin(r2 * omj),
                                           0.0)))
        dots = lax.dot_general(W2, AB, (((1,), (1,)), ((), ())),
                               preferred_element_type=jnp.float32,
                               precision=lax.Precision.HIGHEST)
        t = jnp.sum(coefT * dots, axis=0)
        out_ref[:] = t * (1.0 / (2.0 * Kf * Kf)) + bias_ref[:] + mu_ref[0]

    return pl.pallas_call(
        body,
        grid=(Bn // M,),
        in_specs=[
            pl.BlockSpec(memory_space=pltpu.SMEM),
            pl.BlockSpec((M, D), lambda i: (i, 0)),
            pl.BlockSpec((KP, M), lambda i: (0, i)),
            pl.BlockSpec((M,), lambda i: (i,)),
        ],
        out_specs=pl.BlockSpec((M,), lambda i: (i,)),
        out_shape=jax.ShapeDtypeStruct((Bn,), jnp.float32),
    )(mu_arr, ab, ph, bias)


def kernel(u, it, A, B, shiftA, shiftB, bu, bi, mu):
    K = shiftA.shape[0]
    ab, ph, bias = _sc_gather(
        u.astype(jnp.int32), it.astype(jnp.int32), A, B,
        shiftA.reshape(-1), shiftB.reshape(-1), bu, bi, K)
    return _tc_combine(ab, ph, bias, jnp.reshape(mu, (1,)), K)


# speedup vs baseline: 7.1612x; 1.0248x over previous
import functools

import jax
import jax.numpy as jnp
from jax import lax
from jax.experimental import pallas as pl
from jax.experimental.pallas import tpu as pltpu
from jax.experimental.pallas import tpu_sc as plsc

_NC, _NS, _L = 2, 16, 16
_NW = _NC * _NS
_RCH = 128
_SCH = 128


def _sc_gather(u, it, A, B, sA_flat, sB_flat, bu, bi, K):
    Bn = u.shape[0]
    D = A.shape[1]
    NU = sA_flat.shape[0] // K
    NI = sB_flat.shape[0] // K
    KP = 4 * K
    bpw = Bn // _NW
    nrch = bpw // _RCH
    nsch = bpw // _SCH
    half = D / 2.0
    mesh = plsc.VectorSubcoreMesh(core_axis_name="c", subcore_axis_name="s")

    @functools.partial(
        pl.kernel,
        out_type=(
            jax.ShapeDtypeStruct((Bn, D), jnp.float32),
            jax.ShapeDtypeStruct((KP, Bn), jnp.float32),
            jax.ShapeDtypeStruct((Bn,), jnp.float32),
        ),
        mesh=mesh,
        scratch_types=[
            pltpu.VMEM((bpw,), jnp.int32),
            pltpu.VMEM((bpw,), jnp.int32),
            pltpu.VMEM((2 * K, bpw), jnp.int32),
            pltpu.VMEM((2, _RCH, 128), jnp.float32),
            pltpu.VMEM((2, _RCH, 128), jnp.float32),
            pltpu.VMEM((2, _RCH, 128), jnp.float32),
            pltpu.VMEM((2 * K, bpw), jnp.float32),
            pltpu.VMEM((2, bpw), jnp.float32),
            pltpu.VMEM((3 * K, bpw), jnp.float32),
            pltpu.VMEM((bpw,), jnp.float32),
            pltpu.SemaphoreType.DMA,
            pltpu.SemaphoreType.DMA,
            pltpu.SemaphoreType.DMA,
            pltpu.SemaphoreType.DMA,
            pltpu.SemaphoreType.DMA,
        ],
    )
    def body(u_hbm, it_hbm, A_hbm, B_hbm, sA_hbm, sB_hbm, bu_hbm, bi_hbm,
             ab_out, ph_out, bias_out,
             uidx, iidx, sidx, bufA, bufB, bufP, shbuf, bland, phbuf, ztmp,
             semS, semA, semB, semW, semP):
        wid = lax.axis_index("s") * _NC + lax.axis_index("c")
        base = wid * bpw
        pltpu.sync_copy(u_hbm.at[pl.ds(base, bpw)], uidx)
        pltpu.sync_copy(it_hbm.at[pl.ds(base, bpw)], iidx)

        for t in range(2):
            idxr = uidx if t == 0 else iidx
            N = NU if t == 0 else NI

            def mkidx(p, _):
                k = p >> 5
                j = p & 31
                sl = pl.ds(j * _L, _L)
                sidx[t * K + k, sl] = idxr[sl] + k * N
                return 0
            lax.fori_loop(0, K * (bpw // _L), mkidx, 0)
        for t in range(2):
            tab = sA_hbm if t == 0 else sB_hbm

            def fire_scalar(p, _):
                k = p >> 2
                c = p & 3
                pltpu.make_async_copy(
                    tab.at[sidx.at[t * K + k, pl.ds(c * _SCH, _SCH)]],
                    shbuf.at[t * K + k, pl.ds(c * _SCH, _SCH)],
                    semS).start()
                return 0
            lax.fori_loop(0, K * nsch, fire_scalar, 0)
        for t in range(2):
            tab = bu_hbm if t == 0 else bi_hbm
            idxr = uidx if t == 0 else iidx

            def fire_bias(c, _):
                pltpu.make_async_copy(
                    tab.at[idxr.at[pl.ds(c * _SCH, _SCH)]],
                    bland.at[t, pl.ds(c * _SCH, _SCH)], semS).start()
                return 0
            lax.fori_loop(0, nsch, fire_bias, 0)

        def fire_row(c):
            s = c % 2
            da = pltpu.make_async_copy(
                A_hbm.at[uidx.at[pl.ds(c * _RCH, _RCH)]], bufA.at[s], semA)
            db = pltpu.make_async_copy(
                B_hbm.at[iidx.at[pl.ds(c * _RCH, _RCH)]], bufB.at[s], semB)
            da.start()
            db.start()
            return da, db

        rdescs = {}
        rdescs[0] = fire_row(0)
        if nrch > 1:
            rdescs[1] = fire_row(1)
        wdescs = {}
        for c in range(nrch):
            s = c % 2
            da, db = rdescs.pop(c)
            da.wait()
            db.wait()
            if c - 2 in wdescs:
                wdescs.pop(c - 2).wait()

            def prod(i, _):
                for j in range(D // _L):
                    sl = pl.ds(j * _L, _L)
                    bufP[s, i, sl] = bufA[s, i, sl] * bufB[s, i, sl]
                return 0
            lax.fori_loop(0, _RCH, prod, 0)
            dw = pltpu.make_async_copy(
                bufP.at[s], ab_out.at[pl.ds(base + c * _RCH, _RCH), :], semW)
            dw.start()
            wdescs[c] = dw
            if c + 2 < nrch:
                rdescs[c + 2] = fire_row(c + 2)

        for t in range(2):
            tab = sA_hbm if t == 0 else sB_hbm

            def drain_scalar(p, _):
                k = p >> 2
                c = p & 3
                pltpu.make_async_copy(
                    tab.at[sidx.at[t * K + k, pl.ds(c * _SCH, _SCH)]],
                    shbuf.at[t * K + k, pl.ds(c * _SCH, _SCH)],
                    semS).wait()
                return 0
            lax.fori_loop(0, K * nsch, drain_scalar, 0)
        for t in range(2):
            tab = bu_hbm if t == 0 else bi_hbm
            idxr = uidx if t == 0 else iidx

            def drain_bias(c, _):
                pltpu.make_async_copy(
                    tab.at[idxr.at[pl.ds(c * _SCH, _SCH)]],
                    bland.at[t, pl.ds(c * _SCH, _SCH)], semS).wait()
                return 0
            lax.fori_loop(0, nsch, drain_bias, 0)

        def mkph(k, _):
            sc_ = (k.astype(jnp.float32) + 1.0) * (half / K)

            def inner(j, _2):
                sl = pl.ds(j * _L, _L)
                a = shbuf[k, sl] * sc_
                b = shbuf[K + k, sl] * sc_
                phbuf[k, sl] = a - b
                phbuf[K + k, sl] = a + b
                phbuf[2 * K + k, sl] = a + b
                return 0
            lax.fori_loop(0, bpw // _L, inner, 0)
            return 0
        lax.fori_loop(0, K, mkph, 0)

        def fire_ph(r, _):
            pltpu.make_async_copy(phbuf.at[r],
                                  ph_out.at[r, pl.ds(base, bpw)],
                                  semP).start()
            return 0
        lax.fori_loop(0, 3 * K, fire_ph, 0)

        def zrow(j, _):
            ztmp[pl.ds(j * _L, _L)] = jnp.zeros((_L,), jnp.float32)
            return 0
        lax.fori_loop(0, bpw // _L, zrow, 0)

        def fire_z(k, _):
            pltpu.make_async_copy(ztmp,
                                  ph_out.at[3 * K + k, pl.ds(base, bpw)],
                                  semP).start()
            return 0
        lax.fori_loop(0, K, fire_z, 0)

        def mkbias(j, _):
            sl = pl.ds(j * _L, _L)
            bland[0, sl] = bland[0, sl] + bland[1, sl]
            return 0
        lax.fori_loop(0, bpw // _L, mkbias, 0)
        dbias = pltpu.make_async_copy(bland.at[0],
                                      bias_out.at[pl.ds(base, bpw)], semP)
        dbias.start()

        def drain_ph(r, _):
            pltpu.make_async_copy(phbuf.at[0],
                                  ph_out.at[r, pl.ds(base, bpw)],
                                  semP).wait()
            return 0
        lax.fori_loop(0, KP, drain_ph, 0)
        dbias.wait()
        for c in sorted(wdescs):
            wdescs.pop(c).wait()

    return body(u, it, A, B, sA_flat, sB_flat, bu, bi)


def _tc_combine(ab, ph, bias, mu_arr, K):
    Bn, D = ab.shape
    KP = ph.shape[0]
    M = 4096 if Bn % 4096 == 0 else Bn
    Kf = float(K)
    PI = 3.14159265358979323846

    def body(mu_ref, ab_ref, ph_ref, bias_ref, out_ref, w2_ref):
        @pl.when(pl.program_id(0) == 0)
        def _():
            j2 = lax.broadcasted_iota(jnp.int32, (KP, D), 0)
            r2 = 2.0 * lax.broadcasted_iota(jnp.int32, (KP, D), 1).astype(
                jnp.float32)
            omj = ((j2 % K).astype(jnp.float32) + 1.0) / Kf
            w2_ref[:, :] = jnp.where(
                j2 < K, 1.0,
                jnp.where(j2 < 2 * K, jnp.cos(r2 * omj),
                          jnp.where(j2 < 3 * K, jnp.sin(r2 * omj), 0.0)))

        AB = ab_ref[:, :]
        jr = lax.broadcasted_iota(jnp.int32, (KP, 1), 0)
        off = jnp.where(jr < K, 0.0, jnp.where(jr < 2 * K, PI, PI * 0.5))
        coefT = jnp.cos(ph_ref[:, :] + off)
        dots = lax.dot_general(w2_ref[:, :], AB, (((1,), (1,)), ((), ())),
                               preferred_element_type=jnp.float32,
                               precision=lax.Precision.HIGHEST)
        t = jnp.sum(coefT * dots, axis=0)
        out_ref[:] = t * (1.0 / (2.0 * Kf * Kf)) + bias_ref[:] + mu_ref[0]

    return pl.pallas_call(
        body,
        grid=(Bn // M,),
        in_specs=[
            pl.BlockSpec(memory_space=pltpu.SMEM),
            pl.BlockSpec((M, D), lambda i: (i, 0)),
            pl.BlockSpec((KP, M), lambda i: (0, i)),
            pl.BlockSpec((M,), lambda i: (i,)),
        ],
        out_specs=pl.BlockSpec((M,), lambda i: (i,)),
        out_shape=jax.ShapeDtypeStruct((Bn,), jnp.float32),
        scratch_shapes=[pltpu.VMEM((KP, D), jnp.float32)],
    )(mu_arr, ab, ph, bias)


def kernel(u, it, A, B, shiftA, shiftB, bu, bi, mu):
    K = shiftA.shape[0]
    ab, ph, bias = _sc_gather(
        u.astype(jnp.int32), it.astype(jnp.int32), A, B,
        shiftA.reshape(-1), shiftB.reshape(-1), bu, bi, K)
    return _tc_combine(ab, ph, bias, jnp.reshape(mu, (1,)), K)
